# baseline (speedup 1.0000x reference)
import functools

import jax
import jax.numpy as jnp
from jax import lax
from jax.experimental import pallas as pl
from jax.experimental.pallas import tpu as pltpu
from jax.experimental.pallas import tpu_sc as plsc

NUM_CORES = 2
NUM_SUBCORES = 16
LANES = 16
NW = NUM_CORES * NUM_SUBCORES

BATCH = 16384
EMB_DIM = 32
NPAIR = EMB_DIM // 2
NROWS = 1000000
ROWS_PER_W = BATCH // NW
CHUNK = 128
NCHUNK = ROWS_PER_W // CHUNK
GROUPS = ROWS_PER_W // LANES

W = 65536
NBLK = 16
TGROUPS = EMB_DIM // 16
FLAT = TGROUPS * NBLK * 8 * W


def _pack_body(x_ref, o_ref):
    xb = x_ref[...].astype(jnp.bfloat16)
    y = xb.reshape(2, 4, 2, W)
    lo = jax.lax.bitcast_convert_type(y[:, :, 0, :], jnp.uint16)
    hi = jax.lax.bitcast_convert_type(y[:, :, 1, :], jnp.uint16)
    packed = jax.lax.bitcast_convert_type(
        lo.astype(jnp.uint32) | (hi.astype(jnp.uint32) << 16), jnp.int32)
    packed = packed.reshape(8, W)
    o_ref[...] = packed.reshape(8, W // 128, 128).swapaxes(0, 1).reshape(8 * W)


_pack = pl.pallas_call(
    _pack_body,
    grid=(TGROUPS, NBLK),
    in_specs=[pl.BlockSpec((16, W), lambda t, c: (t, c))],
    out_specs=pl.BlockSpec((8 * W,), lambda t, c: (t * NBLK + c,)),
    out_shape=jax.ShapeDtypeStruct((FLAT,), jnp.int32),
)

_mesh = plsc.VectorSubcoreMesh(core_axis_name="c", subcore_axis_name="s")


@functools.partial(
    pl.kernel,
    out_type=(
        jax.ShapeDtypeStruct((BATCH,), jnp.float32),
        jax.ShapeDtypeStruct((BATCH,), jnp.float32),
    ),
    mesh=_mesh,
    compiler_params=pltpu.CompilerParams(needs_layout_passes=False),
    scratch_types=[
        pltpu.VMEM((NCHUNK, CHUNK), jnp.int32),
        pltpu.VMEM((NCHUNK, CHUNK), jnp.int32),
        pltpu.VMEM((NCHUNK, CHUNK), jnp.int32),
        pltpu.VMEM((NPAIR * NCHUNK, CHUNK), jnp.int32),
        pltpu.VMEM((NPAIR * NCHUNK, CHUNK), jnp.int32),
        pltpu.VMEM((NPAIR * NCHUNK, CHUNK), jnp.int32),
        pltpu.VMEM((NPAIR * ROWS_PER_W,), jnp.int32),
        pltpu.VMEM((NPAIR * ROWS_PER_W,), jnp.int32),
        pltpu.VMEM((NPAIR * ROWS_PER_W,), jnp.int32),
        pltpu.VMEM((ROWS_PER_W,), jnp.float32),
        pltpu.VMEM((ROWS_PER_W,), jnp.float32),
        pltpu.SemaphoreType.DMA,
    ],
)
def _mfbpr_sc(users_hbm, pos_hbm, neg_hbm, ut_hbm, it_hbm,
              pos_out, neg_out,
              uidx, pidx, nidx, ufidx, pfidx, nfidx,
              ubuf, pbuf, nbuf, psc, nsc, sem):
    wid = lax.axis_index("s") * NUM_CORES + lax.axis_index("c")
    blk = wid * NCHUNK

    pltpu.sync_copy(users_hbm.at[pl.ds(blk, NCHUNK)], uidx)
    pltpu.sync_copy(pos_hbm.at[pl.ds(blk, NCHUNK)], pidx)
    pltpu.sync_copy(neg_hbm.at[pl.ds(blk, NCHUNK)], nidx)

    def flat_base(u):
        return ((u >> 16) * (8 * W)
                + ((u >> 7) & 511) * 1024
                + (u & 127))

    def mkidx(i, c):
        for j in range(NCHUNK):
            sl = pl.ds(i * LANES, LANES)
            ub = flat_base(uidx[j, sl])
            pb = flat_base(pidx[j, sl])
            nb = flat_base(nidx[j, sl])

            def step(m, c2):
                moff = ((m >> 3) * (NBLK * 8 * W)
                        + ((m >> 2) & 1) * 512
                        + (m & 3) * 128)
                row = m * NCHUNK + j
                ufidx[row, sl] = ub + moff
                pfidx[row, sl] = pb + moff
                nfidx[row, sl] = nb + moff
                return c2

            lax.fori_loop(0, NPAIR, step, c)
        return c

    lax.fori_loop(0, CHUNK // LANES, mkidx, 0)

    copies = []
    for j in range(NCHUNK):
        for m in range(NPAIR):
            row = m * NCHUNK + j
            dst = pl.ds(m * ROWS_PER_W + j * CHUNK, CHUNK)
            copies.append(
                pltpu.async_copy(ut_hbm.at[ufidx.at[row]], ubuf.at[dst], sem))
            copies.append(
                pltpu.async_copy(it_hbm.at[pfidx.at[row]], pbuf.at[dst], sem))
            copies.append(
                pltpu.async_copy(it_hbm.at[nfidx.at[row]], nbuf.at[dst], sem))
    for cp in copies:
        cp.wait()

    himask = jnp.full((LANES,), jnp.int32(-65536))

    def unpack(v):
        flo = plsc.bitcast(lax.shift_left(v, 16), jnp.float32)
        fhi = plsc.bitcast(v & himask, jnp.float32)
        return flo, fhi

    def group(g, c):
        accp = jnp.zeros((LANES,), jnp.float32)
        accn = jnp.zeros((LANES,), jnp.float32)
        for m in range(NPAIR):
            sl = pl.ds(m * ROWS_PER_W + g * LANES, LANES)
            ulo, uhi = unpack(ubuf[sl])
            plo, phi = unpack(pbuf[sl])
            nlo, nhi = unpack(nbuf[sl])
            accp = accp + ulo * plo + uhi * phi
            accn = accn + ulo * nlo + uhi * nhi
        psc[pl.ds(g * LANES, LANES)] = accp
        nsc[pl.ds(g * LANES, LANES)] = accn
        return c

    lax.fori_loop(0, GROUPS, group, 0)

    base = wid * ROWS_PER_W
    pltpu.sync_copy(psc, pos_out.at[pl.ds(base, ROWS_PER_W)])
    pltpu.sync_copy(nsc, neg_out.at[pl.ds(base, ROWS_PER_W)])


def kernel(users, pos_items, neg_items, user_table, item_table):
    u = users.astype(jnp.int32).reshape(NW * NCHUNK, CHUNK)
    p = pos_items.astype(jnp.int32).reshape(NW * NCHUNK, CHUNK)
    n = neg_items.astype(jnp.int32).reshape(NW * NCHUNK, CHUNK)
    upk = _pack(user_table.T)
    ipk = _pack(item_table.T)
    return _mfbpr_sc(u, p, n, upk, ipk)


# speedup vs baseline: 2.1476x; 2.1476x over previous
import functools

import jax
import jax.numpy as jnp
from jax import lax
from jax.experimental import pallas as pl
from jax.experimental.pallas import tpu as pltpu
from jax.experimental.pallas import tpu_sc as plsc

NUM_CORES = 2
NUM_SUBCORES = 16
LANES = 16
NW = NUM_CORES * NUM_SUBCORES

BATCH = 16384
EMB_DIM = 32
NPAIR = EMB_DIM // 2
NROWS = 1000000
ROWS_PER_W = BATCH // NW
CHUNK = 128
NCHUNK = ROWS_PER_W // CHUNK
GROUPS = ROWS_PER_W // LANES

W = 65536
NBLK = 16
TGROUPS = EMB_DIM // 16
FLAT = TGROUPS * NBLK * 8 * W


def _pack_body(x_ref, o_ref):
    lo = jax.lax.bitcast_convert_type(
        x_ref[0:8, :].astype(jnp.bfloat16), jnp.uint16)
    hi = jax.lax.bitcast_convert_type(
        x_ref[8:16, :].astype(jnp.bfloat16), jnp.uint16)
    packed = jax.lax.bitcast_convert_type(
        lo.astype(jnp.uint32) | (hi.astype(jnp.uint32) << 16), jnp.int32)
    o_ref[...] = packed.reshape(8, W // 128, 128).swapaxes(0, 1).reshape(8 * W)


_pack = pl.pallas_call(
    _pack_body,
    grid=(TGROUPS, NBLK),
    in_specs=[pl.BlockSpec((16, W), lambda t, c: (t, c))],
    out_specs=pl.BlockSpec((8 * W,), lambda t, c: (t * NBLK + c,)),
    out_shape=jax.ShapeDtypeStruct((FLAT,), jnp.int32),
)

_mesh = plsc.VectorSubcoreMesh(core_axis_name="c", subcore_axis_name="s")


@functools.partial(
    pl.kernel,
    out_type=(
        jax.ShapeDtypeStruct((BATCH,), jnp.float32),
        jax.ShapeDtypeStruct((BATCH,), jnp.float32),
    ),
    mesh=_mesh,
    compiler_params=pltpu.CompilerParams(needs_layout_passes=False),
    scratch_types=[
        pltpu.VMEM((NCHUNK, CHUNK), jnp.int32),
        pltpu.VMEM((NCHUNK, CHUNK), jnp.int32),
        pltpu.VMEM((NCHUNK, CHUNK), jnp.int32),
        pltpu.VMEM((NPAIR * NCHUNK, CHUNK), jnp.int32),
        pltpu.VMEM((NPAIR * NCHUNK, CHUNK), jnp.int32),
        pltpu.VMEM((NPAIR * NCHUNK, CHUNK), jnp.int32),
        pltpu.VMEM((NPAIR * ROWS_PER_W,), jnp.int32),
        pltpu.VMEM((NPAIR * ROWS_PER_W,), jnp.int32),
        pltpu.VMEM((NPAIR * ROWS_PER_W,), jnp.int32),
        pltpu.VMEM((ROWS_PER_W,), jnp.float32),
        pltpu.VMEM((ROWS_PER_W,), jnp.float32),
        pltpu.SemaphoreType.DMA,
    ],
)
def _mfbpr_sc(users_hbm, pos_hbm, neg_hbm, ut_hbm, it_hbm,
              pos_out, neg_out,
              uidx, pidx, nidx, ufidx, pfidx, nfidx,
              ubuf, pbuf, nbuf, psc, nsc, sem):
    wid = lax.axis_index("s") * NUM_CORES + lax.axis_index("c")
    blk = wid * NCHUNK

    pltpu.sync_copy(users_hbm.at[pl.ds(blk, NCHUNK)], uidx)
    pltpu.sync_copy(pos_hbm.at[pl.ds(blk, NCHUNK)], pidx)
    pltpu.sync_copy(neg_hbm.at[pl.ds(blk, NCHUNK)], nidx)

    def flat_base(u):
        return ((u >> 16) * (8 * W)
                + ((u >> 7) & 511) * 1024
                + (u & 127))

    def mkidx(i, c):
        for j in range(NCHUNK):
            sl = pl.ds(i * LANES, LANES)
            ub = flat_base(uidx[j, sl])
            pb = flat_base(pidx[j, sl])
            nb = flat_base(nidx[j, sl])

            def step(m, c2):
                moff = (m >> 3) * (NBLK * 8 * W) + (m & 7) * 128
                row = m * NCHUNK + j
                ufidx[row, sl] = ub + moff
                pfidx[row, sl] = pb + moff
                nfidx[row, sl] = nb + moff
                return c2

            lax.fori_loop(0, NPAIR, step, c)
        return c

    lax.fori_loop(0, CHUNK // LANES, mkidx, 0)

    copies = []
    for j in range(NCHUNK):
        for m in range(NPAIR):
            row = m * NCHUNK + j
            dst = pl.ds(m * ROWS_PER_W + j * CHUNK, CHUNK)
            copies.append(
                pltpu.async_copy(ut_hbm.at[ufidx.at[row]], ubuf.at[dst], sem))
            copies.append(
                pltpu.async_copy(it_hbm.at[pfidx.at[row]], pbuf.at[dst], sem))
            copies.append(
                pltpu.async_copy(it_hbm.at[nfidx.at[row]], nbuf.at[dst], sem))
    for cp in copies:
        cp.wait()

    himask = jnp.full((LANES,), jnp.int32(-65536))

    def unpack(v):
        flo = plsc.bitcast(lax.shift_left(v, 16), jnp.float32)
        fhi = plsc.bitcast(v & himask, jnp.float32)
        return flo, fhi

    def group(g, c):
        accp = jnp.zeros((LANES,), jnp.float32)
        accn = jnp.zeros((LANES,), jnp.float32)
        for m in range(NPAIR):
            sl = pl.ds(m * ROWS_PER_W + g * LANES, LANES)
            ulo, uhi = unpack(ubuf[sl])
            plo, phi = unpack(pbuf[sl])
            nlo, nhi = unpack(nbuf[sl])
            accp = accp + ulo * plo + uhi * phi
            accn = accn + ulo * nlo + uhi * nhi
        psc[pl.ds(g * LANES, LANES)] = accp
        nsc[pl.ds(g * LANES, LANES)] = accn
        return c

    lax.fori_loop(0, GROUPS, group, 0)

    base = wid * ROWS_PER_W
    pltpu.sync_copy(psc, pos_out.at[pl.ds(base, ROWS_PER_W)])
    pltpu.sync_copy(nsc, neg_out.at[pl.ds(base, ROWS_PER_W)])


def kernel(users, pos_items, neg_items, user_table, item_table):
    u = users.astype(jnp.int32).reshape(NW * NCHUNK, CHUNK)
    p = pos_items.astype(jnp.int32).reshape(NW * NCHUNK, CHUNK)
    n = neg_items.astype(jnp.int32).reshape(NW * NCHUNK, CHUNK)
    upk = _pack(user_table.T)
    ipk = _pack(item_table.T)
    return _mfbpr_sc(u, p, n, upk, ipk)


# speedup vs baseline: 2.1891x; 1.0194x over previous
import functools

import jax
import jax.numpy as jnp
from jax import lax
from jax.experimental import pallas as pl
from jax.experimental.pallas import tpu as pltpu
from jax.experimental.pallas import tpu_sc as plsc

NUM_CORES = 2
NUM_SUBCORES = 16
LANES = 16
NW = NUM_CORES * NUM_SUBCORES

BATCH = 16384
EMB_DIM = 32
NPAIR = EMB_DIM // 2
NROWS = 1000000
ROWS_PER_W = BATCH // NW
CHUNK = 128
NCHUNK = ROWS_PER_W // CHUNK
GROUPS = ROWS_PER_W // LANES

W = 65536
NBLK = 16
TGROUPS = EMB_DIM // 16
FLAT = TGROUPS * NBLK * 8 * W


def _pack_body(x_ref, o_ref):
    lo = jax.lax.bitcast_convert_type(
        x_ref[0:8, :].astype(jnp.bfloat16), jnp.uint16)
    hi = jax.lax.bitcast_convert_type(
        x_ref[8:16, :].astype(jnp.bfloat16), jnp.uint16)
    packed = jax.lax.bitcast_convert_type(
        lo.astype(jnp.uint32) | (hi.astype(jnp.uint32) << 16), jnp.int32)
    o_ref[...] = packed.reshape(8, W // 128, 128).swapaxes(0, 1).reshape(8 * W)


_pack = pl.pallas_call(
    _pack_body,
    grid=(TGROUPS, NBLK),
    in_specs=[pl.BlockSpec((16, W), lambda t, c: (t, c))],
    out_specs=pl.BlockSpec((8 * W,), lambda t, c: (t * NBLK + c,)),
    out_shape=jax.ShapeDtypeStruct((FLAT,), jnp.int32),
)

_mesh = plsc.VectorSubcoreMesh(core_axis_name="c", subcore_axis_name="s")


def _flat_base(u):
    return (u >> 16) * (8 * W) + ((u >> 7) & 511) * 1024 + (u & 127)


def _pair_off(m):
    return (m >> 3) * (NBLK * 8 * W) + (m & 7) * 128


def _build_idx(idx2d, fidx, i):
    for j in range(NCHUNK):
        sl = pl.ds(i * LANES, LANES)
        base = _flat_base(idx2d[j, sl])

        def step(m, c2):
            fidx[m * NCHUNK + j, sl] = base + _pair_off(m)
            return c2

        lax.fori_loop(0, NPAIR, step, 0)


@functools.partial(
    pl.kernel,
    out_type=jax.ShapeDtypeStruct((NPAIR * BATCH,), jnp.int32),
    mesh=_mesh,
    compiler_params=pltpu.CompilerParams(needs_layout_passes=False),
    scratch_types=[
        pltpu.VMEM((NCHUNK, CHUNK), jnp.int32),
        pltpu.VMEM((NPAIR * NCHUNK, CHUNK), jnp.int32),
        pltpu.VMEM((NPAIR * ROWS_PER_W,), jnp.int32),
        pltpu.SemaphoreType.DMA,
    ],
)
def _user_gather_sc(users_hbm, ut_hbm, upairs_out, uidx, ufidx, ubuf, sem):
    wid = lax.axis_index("s") * NUM_CORES + lax.axis_index("c")
    pltpu.sync_copy(users_hbm.at[pl.ds(wid * NCHUNK, NCHUNK)], uidx)

    lax.fori_loop(0, CHUNK // LANES,
                  lambda i, c: (_build_idx(uidx, ufidx, i), c)[1], 0)

    copies = []
    for j in range(NCHUNK):
        for m in range(NPAIR):
            row = m * NCHUNK + j
            dst = pl.ds(m * ROWS_PER_W + j * CHUNK, CHUNK)
            copies.append(
                pltpu.async_copy(ut_hbm.at[ufidx.at[row]], ubuf.at[dst], sem))
    for cp in copies:
        cp.wait()

    for m in range(NPAIR):
        pltpu.sync_copy(
            ubuf.at[pl.ds(m * ROWS_PER_W, ROWS_PER_W)],
            upairs_out.at[pl.ds(m * BATCH + wid * ROWS_PER_W, ROWS_PER_W)])


@functools.partial(
    pl.kernel,
    out_type=(
        jax.ShapeDtypeStruct((BATCH,), jnp.float32),
        jax.ShapeDtypeStruct((BATCH,), jnp.float32),
    ),
    mesh=_mesh,
    compiler_params=pltpu.CompilerParams(needs_layout_passes=False),
    scratch_types=[
        pltpu.VMEM((NCHUNK, CHUNK), jnp.int32),
        pltpu.VMEM((NCHUNK, CHUNK), jnp.int32),
        pltpu.VMEM((NPAIR * NCHUNK, CHUNK), jnp.int32),
        pltpu.VMEM((NPAIR * NCHUNK, CHUNK), jnp.int32),
        pltpu.VMEM((NPAIR * ROWS_PER_W,), jnp.int32),
        pltpu.VMEM((NPAIR * ROWS_PER_W,), jnp.int32),
        pltpu.VMEM((NPAIR * ROWS_PER_W,), jnp.int32),
        pltpu.VMEM((ROWS_PER_W,), jnp.float32),
        pltpu.VMEM((ROWS_PER_W,), jnp.float32),
        pltpu.SemaphoreType.DMA,
    ],
)
def _score_sc(pos_hbm, neg_hbm, it_hbm, upairs_hbm,
              pos_out, neg_out,
              pidx, nidx, pfidx, nfidx, ubuf, pbuf, nbuf, psc, nsc, sem):
    wid = lax.axis_index("s") * NUM_CORES + lax.axis_index("c")
    blk = wid * NCHUNK

    pltpu.sync_copy(pos_hbm.at[pl.ds(blk, NCHUNK)], pidx)
    pltpu.sync_copy(neg_hbm.at[pl.ds(blk, NCHUNK)], nidx)
    ucopies = []
    for m in range(NPAIR):
        ucopies.append(pltpu.async_copy(
            upairs_hbm.at[pl.ds(m * BATCH + wid * ROWS_PER_W, ROWS_PER_W)],
            ubuf.at[pl.ds(m * ROWS_PER_W, ROWS_PER_W)], sem))

    def build(i, c):
        _build_idx(pidx, pfidx, i)
        _build_idx(nidx, nfidx, i)
        return c

    lax.fori_loop(0, CHUNK // LANES, build, 0)

    copies = []
    for j in range(NCHUNK):
        for m in range(NPAIR):
            row = m * NCHUNK + j
            dst = pl.ds(m * ROWS_PER_W + j * CHUNK, CHUNK)
            copies.append(
                pltpu.async_copy(it_hbm.at[pfidx.at[row]], pbuf.at[dst], sem))
            copies.append(
                pltpu.async_copy(it_hbm.at[nfidx.at[row]], nbuf.at[dst], sem))
    for cp in ucopies:
        cp.wait()
    for cp in copies:
        cp.wait()

    himask = jnp.full((LANES,), jnp.int32(-65536))

    def unpack(v):
        flo = plsc.bitcast(lax.shift_left(v, 16), jnp.float32)
        fhi = plsc.bitcast(v & himask, jnp.float32)
        return flo, fhi

    def group(g, c):
        accp = jnp.zeros((LANES,), jnp.float32)
        accn = jnp.zeros((LANES,), jnp.float32)
        for m in range(NPAIR):
            sl = pl.ds(m * ROWS_PER_W + g * LANES, LANES)
            ulo, uhi = unpack(ubuf[sl])
            plo, phi = unpack(pbuf[sl])
            nlo, nhi = unpack(nbuf[sl])
            accp = accp + ulo * plo + uhi * phi
            accn = accn + ulo * nlo + uhi * nhi
        psc[pl.ds(g * LANES, LANES)] = accp
        nsc[pl.ds(g * LANES, LANES)] = accn
        return c

    lax.fori_loop(0, GROUPS, group, 0)

    base = wid * ROWS_PER_W
    pltpu.sync_copy(psc, pos_out.at[pl.ds(base, ROWS_PER_W)])
    pltpu.sync_copy(nsc, neg_out.at[pl.ds(base, ROWS_PER_W)])


def kernel(users, pos_items, neg_items, user_table, item_table):
    u = users.astype(jnp.int32).reshape(NW * NCHUNK, CHUNK)
    p = pos_items.astype(jnp.int32).reshape(NW * NCHUNK, CHUNK)
    n = neg_items.astype(jnp.int32).reshape(NW * NCHUNK, CHUNK)
    upk = _pack(user_table.T)
    upairs = _user_gather_sc(u, upk)
    ipk = _pack(item_table.T)
    return _score_sc(p, n, ipk, upairs)


# speedup vs baseline: 2.2810x; 1.0420x over previous
import functools

import jax
import jax.numpy as jnp
from jax import lax
from jax.experimental import pallas as pl
from jax.experimental.pallas import tpu as pltpu
from jax.experimental.pallas import tpu_sc as plsc

NUM_CORES = 2
NUM_SUBCORES = 16
LANES = 16
NW = NUM_CORES * NUM_SUBCORES

BATCH = 16384
EMB_DIM = 32
NPAIR = EMB_DIM // 2
NROWS = 1000000
ROWS_PER_W = BATCH // NW
CHUNK = 128
NCHUNK = ROWS_PER_W // CHUNK
GROUPS = ROWS_PER_W // LANES

W = 131072
NBLK = 8
TGROUPS = EMB_DIM // 16
FLAT = TGROUPS * NBLK * 8 * W


def _pack_body(x_ref, o_ref):
    lo = jax.lax.bitcast_convert_type(
        x_ref[0:8, :].astype(jnp.bfloat16), jnp.uint16)
    hi = jax.lax.bitcast_convert_type(
        x_ref[8:16, :].astype(jnp.bfloat16), jnp.uint16)
    packed = jax.lax.bitcast_convert_type(
        lo.astype(jnp.uint32) | (hi.astype(jnp.uint32) << 16), jnp.int32)
    o_ref[...] = packed.reshape(8, W // 128, 128).swapaxes(0, 1).reshape(8 * W)


_pack = pl.pallas_call(
    _pack_body,
    grid=(TGROUPS, NBLK),
    in_specs=[pl.BlockSpec((16, W), lambda t, c: (t, c))],
    out_specs=pl.BlockSpec((8 * W,), lambda t, c: (t * NBLK + c,)),
    out_shape=jax.ShapeDtypeStruct((FLAT,), jnp.int32),
)

_mesh = plsc.VectorSubcoreMesh(core_axis_name="c", subcore_axis_name="s")


def _flat_base(u):
    return (u >> 17) * (8 * W) + ((u >> 7) & 1023) * 1024 + (u & 127)


def _pair_off(m):
    return (m >> 3) * (NBLK * 8 * W) + (m & 7) * 128


def _build_idx(idx2d, fidx, i):
    for j in range(NCHUNK):
        sl = pl.ds(i * LANES, LANES)
        base = _flat_base(idx2d[j, sl])

        def step(m, c2):
            fidx[m * NCHUNK + j, sl] = base + _pair_off(m)
            return c2

        lax.fori_loop(0, NPAIR, step, 0)


@functools.partial(
    pl.kernel,
    out_type=jax.ShapeDtypeStruct((NPAIR * BATCH,), jnp.int32),
    mesh=_mesh,
    compiler_params=pltpu.CompilerParams(needs_layout_passes=False),
    scratch_types=[
        pltpu.VMEM((NCHUNK, CHUNK), jnp.int32),
        pltpu.VMEM((NPAIR * NCHUNK, CHUNK), jnp.int32),
        pltpu.VMEM((NPAIR * ROWS_PER_W,), jnp.int32),
        pltpu.SemaphoreType.DMA,
    ],
)
def _user_gather_sc(users_hbm, ut_hbm, upairs_out, uidx, ufidx, ubuf, sem):
    wid = lax.axis_index("s") * NUM_CORES + lax.axis_index("c")
    pltpu.sync_copy(users_hbm.at[pl.ds(wid * NCHUNK, NCHUNK)], uidx)

    lax.fori_loop(0, CHUNK // LANES,
                  lambda i, c: (_build_idx(uidx, ufidx, i), c)[1], 0)

    copies = []
    for j in range(NCHUNK):
        for m in range(NPAIR):
            row = m * NCHUNK + j
            dst = pl.ds(m * ROWS_PER_W + j * CHUNK, CHUNK)
            copies.append(
                pltpu.async_copy(ut_hbm.at[ufidx.at[row]], ubuf.at[dst], sem))
    for cp in copies:
        cp.wait()

    for m in range(NPAIR):
        pltpu.sync_copy(
            ubuf.at[pl.ds(m * ROWS_PER_W, ROWS_PER_W)],
            upairs_out.at[pl.ds(m * BATCH + wid * ROWS_PER_W, ROWS_PER_W)])


@functools.partial(
    pl.kernel,
    out_type=(
        jax.ShapeDtypeStruct((BATCH,), jnp.float32),
        jax.ShapeDtypeStruct((BATCH,), jnp.float32),
    ),
    mesh=_mesh,
    compiler_params=pltpu.CompilerParams(needs_layout_passes=False),
    scratch_types=[
        pltpu.VMEM((NCHUNK, CHUNK), jnp.int32),
        pltpu.VMEM((NCHUNK, CHUNK), jnp.int32),
        pltpu.VMEM((NPAIR * NCHUNK, CHUNK), jnp.int32),
        pltpu.VMEM((NPAIR * NCHUNK, CHUNK), jnp.int32),
        pltpu.VMEM((NPAIR * ROWS_PER_W,), jnp.int32),
        pltpu.VMEM((NPAIR * ROWS_PER_W,), jnp.int32),
        pltpu.VMEM((NPAIR * ROWS_PER_W,), jnp.int32),
        pltpu.VMEM((ROWS_PER_W,), jnp.float32),
        pltpu.VMEM((ROWS_PER_W,), jnp.float32),
        pltpu.SemaphoreType.DMA,
    ],
)
def _score_sc(pos_hbm, neg_hbm, it_hbm, upairs_hbm,
              pos_out, neg_out,
              pidx, nidx, pfidx, nfidx, ubuf, pbuf, nbuf, psc, nsc, sem):
    wid = lax.axis_index("s") * NUM_CORES + lax.axis_index("c")
    blk = wid * NCHUNK

    pltpu.sync_copy(pos_hbm.at[pl.ds(blk, NCHUNK)], pidx)
    pltpu.sync_copy(neg_hbm.at[pl.ds(blk, NCHUNK)], nidx)
    ucopies = []
    for m in range(NPAIR):
        ucopies.append(pltpu.async_copy(
            upairs_hbm.at[pl.ds(m * BATCH + wid * ROWS_PER_W, ROWS_PER_W)],
            ubuf.at[pl.ds(m * ROWS_PER_W, ROWS_PER_W)], sem))

    def build(i, c):
        _build_idx(pidx, pfidx, i)
        _build_idx(nidx, nfidx, i)
        return c

    lax.fori_loop(0, CHUNK // LANES, build, 0)

    copies = []
    for j in range(NCHUNK):
        for m in range(NPAIR):
            row = m * NCHUNK + j
            dst = pl.ds(m * ROWS_PER_W + j * CHUNK, CHUNK)
            copies.append(
                pltpu.async_copy(it_hbm.at[pfidx.at[row]], pbuf.at[dst], sem))
            copies.append(
                pltpu.async_copy(it_hbm.at[nfidx.at[row]], nbuf.at[dst], sem))
    for cp in ucopies:
        cp.wait()
    for cp in copies:
        cp.wait()

    himask = jnp.full((LANES,), jnp.int32(-65536))

    def unpack(v):
        flo = plsc.bitcast(lax.shift_left(v, 16), jnp.float32)
        fhi = plsc.bitcast(v & himask, jnp.float32)
        return flo, fhi

    def group(g, c):
        accp = jnp.zeros((LANES,), jnp.float32)
        accn = jnp.zeros((LANES,), jnp.float32)
        for m in range(NPAIR):
            sl = pl.ds(m * ROWS_PER_W + g * LANES, LANES)
            ulo, uhi = unpack(ubuf[sl])
            plo, phi = unpack(pbuf[sl])
            nlo, nhi = unpack(nbuf[sl])
            accp = accp + ulo * plo + uhi * phi
            accn = accn + ulo * nlo + uhi * nhi
        psc[pl.ds(g * LANES, LANES)] = accp
        nsc[pl.ds(g * LANES, LANES)] = accn
        return c

    lax.fori_loop(0, GROUPS, group, 0)

    base = wid * ROWS_PER_W
    pltpu.sync_copy(psc, pos_out.at[pl.ds(base, ROWS_PER_W)])
    pltpu.sync_copy(nsc, neg_out.at[pl.ds(base, ROWS_PER_W)])


def kernel(users, pos_items, neg_items, user_table, item_table):
    u = users.astype(jnp.int32).reshape(NW * NCHUNK, CHUNK)
    p = pos_items.astype(jnp.int32).reshape(NW * NCHUNK, CHUNK)
    n = neg_items.astype(jnp.int32).reshape(NW * NCHUNK, CHUNK)
    upk = _pack(user_table.T)
    upairs = _user_gather_sc(u, upk)
    ipk = _pack(item_table.T)
    return _score_sc(p, n, ipk, upairs)


# speedup vs baseline: 2.2837x; 1.0012x over previous
import functools

import jax
import jax.numpy as jnp
from jax import lax
from jax.experimental import pallas as pl
from jax.experimental.pallas import tpu as pltpu
from jax.experimental.pallas import tpu_sc as plsc

NUM_CORES = 2
NUM_SUBCORES = 16
LANES = 16
NW = NUM_CORES * NUM_SUBCORES

BATCH = 16384
EMB_DIM = 32
NPAIR = EMB_DIM // 2
NROWS = 1000000
ROWS_PER_W = BATCH // NW
CHUNK = 128
NCHUNK = ROWS_PER_W // CHUNK
GROUPS = ROWS_PER_W // LANES

W = 131072
NBLK = 8
TGROUPS = EMB_DIM // 16
FLAT = TGROUPS * NBLK * 8 * W


def _pack_body(lo_ref, hi_ref, o_ref):
    lo = jax.lax.bitcast_convert_type(
        lo_ref[...].astype(jnp.bfloat16), jnp.uint16)
    hi = jax.lax.bitcast_convert_type(
        hi_ref[...].astype(jnp.bfloat16), jnp.uint16)
    packed = jax.lax.bitcast_convert_type(
        lo.astype(jnp.uint32) | (hi.astype(jnp.uint32) << 16), jnp.int32)
    o_ref[...] = packed.reshape(8, W // 128, 128).swapaxes(0, 1).reshape(8 * W)


_pack = pl.pallas_call(
    _pack_body,
    grid=(TGROUPS, NBLK),
    in_specs=[
        pl.BlockSpec((8, W), lambda t, c: (2 * t, c)),
        pl.BlockSpec((8, W), lambda t, c: (2 * t + 1, c)),
    ],
    out_specs=pl.BlockSpec((8 * W,), lambda t, c: (t * NBLK + c,)),
    out_shape=jax.ShapeDtypeStruct((FLAT,), jnp.int32),
)

_mesh = plsc.VectorSubcoreMesh(core_axis_name="c", subcore_axis_name="s")


def _flat_base(u):
    return (u >> 17) * (8 * W) + ((u >> 7) & 1023) * 1024 + (u & 127)


def _pair_off(m):
    return (m >> 3) * (NBLK * 8 * W) + (m & 7) * 128


def _build_idx(idx2d, fidx, i):
    for j in range(NCHUNK):
        sl = pl.ds(i * LANES, LANES)
        base = _flat_base(idx2d[j, sl])

        def step(m, c2):
            fidx[m * NCHUNK + j, sl] = base + _pair_off(m)
            return c2

        lax.fori_loop(0, NPAIR, step, 0)


@functools.partial(
    pl.kernel,
    out_type=jax.ShapeDtypeStruct((NPAIR * BATCH,), jnp.int32),
    mesh=_mesh,
    compiler_params=pltpu.CompilerParams(needs_layout_passes=False),
    scratch_types=[
        pltpu.VMEM((NCHUNK, CHUNK), jnp.int32),
        pltpu.VMEM((NPAIR * NCHUNK, CHUNK), jnp.int32),
        pltpu.VMEM((NPAIR * ROWS_PER_W,), jnp.int32),
        pltpu.SemaphoreType.DMA,
    ],
)
def _user_gather_sc(users_hbm, ut_hbm, upairs_out, uidx, ufidx, ubuf, sem):
    wid = lax.axis_index("s") * NUM_CORES + lax.axis_index("c")
    pltpu.sync_copy(users_hbm.at[pl.ds(wid * NCHUNK, NCHUNK)], uidx)

    lax.fori_loop(0, CHUNK // LANES,
                  lambda i, c: (_build_idx(uidx, ufidx, i), c)[1], 0)

    copies = []
    for j in range(NCHUNK):
        for m in range(NPAIR):
            row = m * NCHUNK + j
            dst = pl.ds(m * ROWS_PER_W + j * CHUNK, CHUNK)
            copies.append(
                pltpu.async_copy(ut_hbm.at[ufidx.at[row]], ubuf.at[dst], sem))
    for cp in copies:
        cp.wait()

    for m in range(NPAIR):
        pltpu.sync_copy(
            ubuf.at[pl.ds(m * ROWS_PER_W, ROWS_PER_W)],
            upairs_out.at[pl.ds(m * BATCH + wid * ROWS_PER_W, ROWS_PER_W)])


@functools.partial(
    pl.kernel,
    out_type=(
        jax.ShapeDtypeStruct((BATCH,), jnp.float32),
        jax.ShapeDtypeStruct((BATCH,), jnp.float32),
    ),
    mesh=_mesh,
    compiler_params=pltpu.CompilerParams(needs_layout_passes=False),
    scratch_types=[
        pltpu.VMEM((NCHUNK, CHUNK), jnp.int32),
        pltpu.VMEM((NCHUNK, CHUNK), jnp.int32),
        pltpu.VMEM((NPAIR * NCHUNK, CHUNK), jnp.int32),
        pltpu.VMEM((NPAIR * NCHUNK, CHUNK), jnp.int32),
        pltpu.VMEM((NPAIR * ROWS_PER_W,), jnp.int32),
        pltpu.VMEM((NPAIR * ROWS_PER_W,), jnp.int32),
        pltpu.VMEM((NPAIR * ROWS_PER_W,), jnp.int32),
        pltpu.VMEM((ROWS_PER_W,), jnp.float32),
        pltpu.VMEM((ROWS_PER_W,), jnp.float32),
        pltpu.SemaphoreType.DMA,
    ],
)
def _score_sc(pos_hbm, neg_hbm, it_hbm, upairs_hbm,
              pos_out, neg_out,
              pidx, nidx, pfidx, nfidx, ubuf, pbuf, nbuf, psc, nsc, sem):
    wid = lax.axis_index("s") * NUM_CORES + lax.axis_index("c")
    blk = wid * NCHUNK

    pltpu.sync_copy(pos_hbm.at[pl.ds(blk, NCHUNK)], pidx)
    pltpu.sync_copy(neg_hbm.at[pl.ds(blk, NCHUNK)], nidx)
    ucopies = []
    for m in range(NPAIR):
        ucopies.append(pltpu.async_copy(
            upairs_hbm.at[pl.ds(m * BATCH + wid * ROWS_PER_W, ROWS_PER_W)],
            ubuf.at[pl.ds(m * ROWS_PER_W, ROWS_PER_W)], sem))

    def build(i, c):
        _build_idx(pidx, pfidx, i)
        _build_idx(nidx, nfidx, i)
        return c

    lax.fori_loop(0, CHUNK // LANES, build, 0)

    copies = []
    for j in range(NCHUNK):
        for m in range(NPAIR):
            row = m * NCHUNK + j
            dst = pl.ds(m * ROWS_PER_W + j * CHUNK, CHUNK)
            copies.append(
                pltpu.async_copy(it_hbm.at[pfidx.at[row]], pbuf.at[dst], sem))
            copies.append(
                pltpu.async_copy(it_hbm.at[nfidx.at[row]], nbuf.at[dst], sem))
    for cp in ucopies:
        cp.wait()
    for cp in copies:
        cp.wait()

    himask = jnp.full((LANES,), jnp.int32(-65536))

    def unpack(v):
        flo = plsc.bitcast(lax.shift_left(v, 16), jnp.float32)
        fhi = plsc.bitcast(v & himask, jnp.float32)
        return flo, fhi

    def group(g, c):
        accp = jnp.zeros((LANES,), jnp.float32)
        accn = jnp.zeros((LANES,), jnp.float32)
        for m in range(NPAIR):
            sl = pl.ds(m * ROWS_PER_W + g * LANES, LANES)
            ulo, uhi = unpack(ubuf[sl])
            plo, phi = unpack(pbuf[sl])
            nlo, nhi = unpack(nbuf[sl])
            accp = accp + ulo * plo + uhi * phi
            accn = accn + ulo * nlo + uhi * nhi
        psc[pl.ds(g * LANES, LANES)] = accp
        nsc[pl.ds(g * LANES, LANES)] = accn
        return c

    lax.fori_loop(0, GROUPS, group, 0)

    base = wid * ROWS_PER_W
    pltpu.sync_copy(psc, pos_out.at[pl.ds(base, ROWS_PER_W)])
    pltpu.sync_copy(nsc, neg_out.at[pl.ds(base, ROWS_PER_W)])


def kernel(users, pos_items, neg_items, user_table, item_table):
    u = users.astype(jnp.int32).reshape(NW * NCHUNK, CHUNK)
    p = pos_items.astype(jnp.int32).reshape(NW * NCHUNK, CHUNK)
    n = neg_items.astype(jnp.int32).reshape(NW * NCHUNK, CHUNK)
    ut = user_table.T
    it = item_table.T
    upk = _pack(ut, ut)
    upairs = _user_gather_sc(u, upk)
    ipk = _pack(it, it)
    return _score_sc(p, n, ipk, upairs)


# speedup vs baseline: 2.3122x; 1.0125x over previous
import functools

import jax
import jax.numpy as jnp
from jax import lax
from jax.experimental import pallas as pl
from jax.experimental.pallas import tpu as pltpu
from jax.experimental.pallas import tpu_sc as plsc

NUM_CORES = 2
NUM_SUBCORES = 16
LANES = 16
NW = NUM_CORES * NUM_SUBCORES

BATCH = 16384
EMB_DIM = 32
NPAIR = EMB_DIM // 2
NROWS = 1000000
ROWS_PER_W = BATCH // NW
CHUNK = 128
NCHUNK = ROWS_PER_W // CHUNK
GROUPS = ROWS_PER_W // LANES

W = 131072
NBLK = 8
TGROUPS = EMB_DIM // 16
FLAT = TGROUPS * NBLK * 8 * W


def _pack_body(lo_ref, hi_ref, o_ref):
    lo = jax.lax.bitcast_convert_type(
        lo_ref[...].astype(jnp.bfloat16), jnp.uint16)
    hi = jax.lax.bitcast_convert_type(
        hi_ref[...].astype(jnp.bfloat16), jnp.uint16)
    packed = jax.lax.bitcast_convert_type(
        lo.astype(jnp.uint32) | (hi.astype(jnp.uint32) << 16), jnp.int32)
    o_ref[...] = packed.reshape(8, W // 128, 128).swapaxes(0, 1).reshape(8 * W)


_pack = pl.pallas_call(
    _pack_body,
    grid=(TGROUPS, NBLK),
    in_specs=[
        pl.BlockSpec((8, W), lambda t, c: (2 * t, c)),
        pl.BlockSpec((8, W), lambda t, c: (2 * t + 1, c)),
    ],
    out_specs=pl.BlockSpec((8 * W,), lambda t, c: (t * NBLK + c,)),
    out_shape=jax.ShapeDtypeStruct((FLAT,), jnp.int32),
)

_mesh = plsc.VectorSubcoreMesh(core_axis_name="c", subcore_axis_name="s")


def _flat_base(u):
    return (u >> 17) * (8 * W) + ((u >> 7) & 1023) * 1024 + (u & 127)


def _pair_off(m):
    return (m >> 3) * (NBLK * 8 * W) + (m & 7) * 128


def _build_chunk_idx(idx2d, fidx, j):
    def lane_group(i, c):
        sl = pl.ds(i * LANES, LANES)
        base = _flat_base(idx2d[j, sl])

        def step(m, c2):
            fidx[m * NCHUNK + j, sl] = base + _pair_off(m)
            return c2

        lax.fori_loop(0, NPAIR, step, c)
        return c

    lax.fori_loop(0, CHUNK // LANES, lane_group, 0)


@functools.partial(
    pl.kernel,
    out_type=jax.ShapeDtypeStruct((NPAIR * BATCH,), jnp.int32),
    mesh=_mesh,
    compiler_params=pltpu.CompilerParams(needs_layout_passes=False),
    scratch_types=[
        pltpu.VMEM((NCHUNK, CHUNK), jnp.int32),
        pltpu.VMEM((NPAIR * NCHUNK, CHUNK), jnp.int32),
        pltpu.VMEM((NPAIR * ROWS_PER_W,), jnp.int32),
        pltpu.SemaphoreType.DMA,
    ],
)
def _user_gather_sc(users_hbm, ut_hbm, upairs_out, uidx, ufidx, ubuf, sem):
    wid = lax.axis_index("s") * NUM_CORES + lax.axis_index("c")
    pltpu.sync_copy(users_hbm.at[pl.ds(wid * NCHUNK, NCHUNK)], uidx)

    copies = []
    for j in range(NCHUNK):
        _build_chunk_idx(uidx, ufidx, j)
        for m in range(NPAIR):
            row = m * NCHUNK + j
            dst = pl.ds(m * ROWS_PER_W + j * CHUNK, CHUNK)
            copies.append(
                pltpu.async_copy(ut_hbm.at[ufidx.at[row]], ubuf.at[dst], sem))
    for cp in copies:
        cp.wait()

    for m in range(NPAIR):
        pltpu.sync_copy(
            ubuf.at[pl.ds(m * ROWS_PER_W, ROWS_PER_W)],
            upairs_out.at[pl.ds(m * BATCH + wid * ROWS_PER_W, ROWS_PER_W)])


@functools.partial(
    pl.kernel,
    out_type=(
        jax.ShapeDtypeStruct((BATCH,), jnp.float32),
        jax.ShapeDtypeStruct((BATCH,), jnp.float32),
    ),
    mesh=_mesh,
    compiler_params=pltpu.CompilerParams(needs_layout_passes=False),
    scratch_types=[
        pltpu.VMEM((NCHUNK, CHUNK), jnp.int32),
        pltpu.VMEM((NCHUNK, CHUNK), jnp.int32),
        pltpu.VMEM((NPAIR * NCHUNK, CHUNK), jnp.int32),
        pltpu.VMEM((NPAIR * NCHUNK, CHUNK), jnp.int32),
        pltpu.VMEM((NPAIR * ROWS_PER_W,), jnp.int32),
        pltpu.VMEM((NPAIR * ROWS_PER_W,), jnp.int32),
        pltpu.VMEM((NPAIR * ROWS_PER_W,), jnp.int32),
        pltpu.VMEM((ROWS_PER_W,), jnp.float32),
        pltpu.VMEM((ROWS_PER_W,), jnp.float32),
        pltpu.SemaphoreType.DMA,
    ],
)
def _score_sc(pos_hbm, neg_hbm, it_hbm, upairs_hbm,
              pos_out, neg_out,
              pidx, nidx, pfidx, nfidx, ubuf, pbuf, nbuf, psc, nsc, sem):
    wid = lax.axis_index("s") * NUM_CORES + lax.axis_index("c")
    blk = wid * NCHUNK

    pltpu.sync_copy(pos_hbm.at[pl.ds(blk, NCHUNK)], pidx)
    pltpu.sync_copy(neg_hbm.at[pl.ds(blk, NCHUNK)], nidx)
    ucopies = []
    for m in range(NPAIR):
        ucopies.append(pltpu.async_copy(
            upairs_hbm.at[pl.ds(m * BATCH + wid * ROWS_PER_W, ROWS_PER_W)],
            ubuf.at[pl.ds(m * ROWS_PER_W, ROWS_PER_W)], sem))

    copies = []
    for j in range(NCHUNK):
        _build_chunk_idx(pidx, pfidx, j)
        _build_chunk_idx(nidx, nfidx, j)
        for m in range(NPAIR):
            row = m * NCHUNK + j
            dst = pl.ds(m * ROWS_PER_W + j * CHUNK, CHUNK)
            copies.append(
                pltpu.async_copy(it_hbm.at[pfidx.at[row]], pbuf.at[dst], sem))
            copies.append(
                pltpu.async_copy(it_hbm.at[nfidx.at[row]], nbuf.at[dst], sem))
    for cp in ucopies:
        cp.wait()
    for cp in copies:
        cp.wait()

    himask = jnp.full((LANES,), jnp.int32(-65536))

    def unpack(v):
        flo = plsc.bitcast(lax.shift_left(v, 16), jnp.float32)
        fhi = plsc.bitcast(v & himask, jnp.float32)
        return flo, fhi

    def group(g, c):
        accp = jnp.zeros((LANES,), jnp.float32)
        accn = jnp.zeros((LANES,), jnp.float32)
        for m in range(NPAIR):
            sl = pl.ds(m * ROWS_PER_W + g * LANES, LANES)
            ulo, uhi = unpack(ubuf[sl])
            plo, phi = unpack(pbuf[sl])
            nlo, nhi = unpack(nbuf[sl])
            accp = accp + ulo * plo + uhi * phi
            accn = accn + ulo * nlo + uhi * nhi
        psc[pl.ds(g * LANES, LANES)] = accp
        nsc[pl.ds(g * LANES, LANES)] = accn
        return c

    lax.fori_loop(0, GROUPS, group, 0)

    base = wid * ROWS_PER_W
    pltpu.sync_copy(psc, pos_out.at[pl.ds(base, ROWS_PER_W)])
    pltpu.sync_copy(nsc, neg_out.at[pl.ds(base, ROWS_PER_W)])


def kernel(users, pos_items, neg_items, user_table, item_table):
    u = users.astype(jnp.int32).reshape(NW * NCHUNK, CHUNK)
    p = pos_items.astype(jnp.int32).reshape(NW * NCHUNK, CHUNK)
    n = neg_items.astype(jnp.int32).reshape(NW * NCHUNK, CHUNK)
    ut = user_table.T
    it = item_table.T
    upk = _pack(ut, ut)
    upairs = _user_gather_sc(u, upk)
    ipk = _pack(it, it)
    return _score_sc(p, n, ipk, upairs)
